# baseline (speedup 1.0000x reference)
import functools

import jax
import jax.numpy as jnp
from jax.experimental import pallas as pl
from jax.experimental.pallas import tpu as pltpu

_BN = 400


def _gcn_body(x_ref, s_ref, w_ref, b_ref, o_ref, c_ref):
    i = pl.program_id(0)

    @pl.when(i == 0)
    def _():
        c32 = jax.lax.dot_general(
            s_ref[:], w_ref[:], (((1,), (1,)), ((), ())),
            preferred_element_type=jnp.float32)
        c_ref[:] = c32.astype(jnp.bfloat16)

    x_bf = x_ref[:].astype(jnp.bfloat16)
    acc = jnp.dot(x_bf, c_ref[:], preferred_element_type=jnp.float32)
    o_ref[:] = jnp.maximum(acc + b_ref[:], 0.0)


@functools.partial(jax.jit, static_argnames=())
def kernel(x, support, W, b):
    n, d = x.shape
    out_c, in_c = W.shape
    bn = _BN
    out = pl.pallas_call(
        _gcn_body,
        grid=(n // bn,),
        in_specs=[
            pl.BlockSpec((bn, d), lambda i: (i, 0)),
            pl.BlockSpec((d, in_c), lambda i: (0, 0)),
            pl.BlockSpec((out_c, in_c), lambda i: (0, 0)),
            pl.BlockSpec((1, out_c), lambda i: (0, 0)),
        ],
        out_specs=pl.BlockSpec((bn, out_c), lambda i: (i, 0)),
        out_shape=jax.ShapeDtypeStruct((n, out_c), jnp.float32),
        scratch_shapes=[pltpu.VMEM((d, out_c), jnp.bfloat16)],
    )(x, support, W, b.reshape(1, out_c))
    return out


# speedup vs baseline: 1.5657x; 1.5657x over previous
import functools

import jax
import jax.numpy as jnp
from jax.experimental import pallas as pl
from jax.experimental.pallas import tpu as pltpu

_BN = 2000


def _gcn_body(x_ref, s_ref, w_ref, b_ref, o_ref, c_ref):
    i = pl.program_id(0)

    @pl.when(i == 0)
    def _():
        c32 = jax.lax.dot_general(
            s_ref[:], w_ref[:], (((1,), (1,)), ((), ())),
            preferred_element_type=jnp.float32)
        c_ref[:] = c32.astype(jnp.bfloat16)

    x_bf = x_ref[:].astype(jnp.bfloat16)
    acc = jnp.dot(x_bf, c_ref[:], preferred_element_type=jnp.float32)
    o_ref[:] = jnp.maximum(acc + b_ref[:], 0.0)


@functools.partial(jax.jit, static_argnames=())
def kernel(x, support, W, b):
    n, d = x.shape
    out_c, in_c = W.shape
    bn = _BN
    out = pl.pallas_call(
        _gcn_body,
        grid=(n // bn,),
        in_specs=[
            pl.BlockSpec((bn, d), lambda i: (i, 0)),
            pl.BlockSpec((d, in_c), lambda i: (0, 0)),
            pl.BlockSpec((out_c, in_c), lambda i: (0, 0)),
            pl.BlockSpec((1, out_c), lambda i: (0, 0)),
        ],
        out_specs=pl.BlockSpec((bn, out_c), lambda i: (i, 0)),
        out_shape=jax.ShapeDtypeStruct((n, out_c), jnp.float32),
        scratch_shapes=[pltpu.VMEM((d, out_c), jnp.bfloat16)],
        compiler_params=pltpu.CompilerParams(
            vmem_limit_bytes=120 * 1024 * 1024),
    )(x, support, W, b.reshape(1, out_c))
    return out
